# baseline (speedup 1.0000x reference)
import functools

import jax
import jax.numpy as jnp
from jax.experimental import pallas as pl
from jax.experimental.pallas import tpu as pltpu
from jax.experimental.pallas import tpu_sc as plsc

T = 2048
M = 2048
E = 8
H = 4096
C = 256
BH = 1024
NH = H // BH

NSLOT = 2560


def _router_body(lg_ref, pos_ref, gate_ref):
    l = lg_ref[...]
    m = jnp.max(l, axis=1, keepdims=True)
    io8 = jax.lax.broadcasted_iota(jnp.int32, (T, E), 1)
    idx = jnp.min(jnp.where(l >= m, io8, E), axis=1, keepdims=True)
    gate = 1.0 / jnp.sum(jnp.exp(l - m), axis=1, keepdims=True)
    onehot = (io8 == idx).astype(jnp.bfloat16)
    CH = 128
    ti = jax.lax.broadcasted_iota(jnp.int32, (CH, CH), 0)
    tj = jax.lax.broadcasted_iota(jnp.int32, (CH, CH), 1)
    tril = (tj < ti).astype(jnp.bfloat16)
    off = jnp.zeros((1, E), jnp.float32)
    pieces = []
    for b in range(T // CH):
        blk = onehot[b * CH:(b + 1) * CH, :]
        cum_b = jnp.dot(tril, blk, preferred_element_type=jnp.float32)
        pieces.append(cum_b + off)
        off = off + jnp.sum(blk.astype(jnp.float32), axis=0, keepdims=True)
    cum = jnp.concatenate(pieces, axis=0)
    loc = jnp.sum(cum * onehot.astype(jnp.float32),
                  axis=1, keepdims=True).astype(jnp.int32)
    pos_ref[...] = jnp.where(loc < C, idx * C + loc, E * C)
    gate_ref[...] = gate


@jax.jit
def _router(logits):
    return pl.pallas_call(
        _router_body,
        out_shape=(jax.ShapeDtypeStruct((T, 1), jnp.int32),
                   jax.ShapeDtypeStruct((T, 1), jnp.float32)),
    )(logits)


def _slots_body(pos_hbm, gate_hbm, ssrc_hbm, sgate_hbm,
                pos_v, gate_v, tid_v, zbuf_i, zbuf_f, obuf_i, obuf_f,
                shared_i, shared_f):
    cid = jax.lax.axis_index("c")
    sid = jax.lax.axis_index("s")

    @pl.when(cid == 0)
    def _():
        base = sid * 128
        pltpu.sync_copy(pos_hbm.at[pl.ds(base, 128)], pos_v)
        pltpu.sync_copy(gate_hbm.at[pl.ds(base, 128)], gate_v)
        for k in range(8):
            tid_v[pl.ds(k * 16, 16)] = (
                jax.lax.iota(jnp.int32, 16) + (base + k * 16 + 1))
        for k in range(10):
            zbuf_i[pl.ds(k * 16, 16)] = jnp.zeros((16,), jnp.int32)
            zbuf_f[pl.ds(k * 16, 16)] = jnp.zeros((16,), jnp.float32)
        zb = sid * (NSLOT // 16)
        pltpu.sync_copy(zbuf_i, shared_i.at[pl.ds(zb, NSLOT // 16)])
        pltpu.sync_copy(zbuf_f, shared_f.at[pl.ds(zb, NSLOT // 16)])

    plsc.subcore_barrier()

    @pl.when(cid == 0)
    def _():
        pltpu.sync_copy(tid_v, shared_i.at[pos_v], add=True)
        pltpu.sync_copy(gate_v, shared_f.at[pos_v], add=True)

    plsc.subcore_barrier()

    @pl.when(cid == 0)
    def _():
        rb = sid * 128
        pltpu.sync_copy(shared_i.at[pl.ds(rb, 128)], ssrc_hbm.at[pl.ds(rb, 128)])
        pltpu.sync_copy(shared_f.at[pl.ds(rb, 128)], sgate_hbm.at[pl.ds(rb, 128)])


@jax.jit
def _slots(pos, gate):
    f = functools.partial(
        pl.kernel,
        out_type=(jax.ShapeDtypeStruct((E * C,), jnp.int32),
                  jax.ShapeDtypeStruct((E * C,), jnp.float32)),
        mesh=plsc.VectorSubcoreMesh(core_axis_name="c", subcore_axis_name="s"),
        scratch_types=[
            pltpu.VMEM((128,), jnp.int32),
            pltpu.VMEM((128,), jnp.float32),
            pltpu.VMEM((128,), jnp.int32),
            pltpu.VMEM((NSLOT // 16,), jnp.int32),
            pltpu.VMEM((NSLOT // 16,), jnp.float32),
            pltpu.VMEM((128,), jnp.int32),
            pltpu.VMEM((128,), jnp.float32),
            pltpu.VMEM_SHARED((NSLOT,), jnp.int32),
            pltpu.VMEM_SHARED((NSLOT,), jnp.float32),
        ],
    )(_slots_body)
    return f(pos, gate)


def _moe_body(x_ref, ssrc_ref, sgate_ref, fc1_ref, b1_ref, fc2_ref, b2_ref,
              out_ref, xe_raw, xs_ref, acc_ref, stage_ref, cnt_ref,
              sem_in, sem_out, sem_z):
    e = pl.program_id(0)
    nh = pl.program_id(1)

    def issue_gather(e1, b):
        def body(c, _):
            t1 = ssrc_ref[e1, c]
            t = jnp.maximum(t1 - 1, 0)
            pltpu.make_async_copy(
                x_ref.at[pl.ds(t, 1), :],
                xe_raw.at[b, pl.ds(c, 1), :],
                sem_in.at[b],
            ).start()
            return 0
        jax.lax.fori_loop(0, C, body, 0, unroll=False)

    @pl.when(nh == 0)
    def _():
        b = jax.lax.rem(e, 2)

        @pl.when(e == 0)
        def _():
            issue_gather(0, 0)
            stage_ref[0] = jnp.zeros_like(stage_ref[0])

            def zbody(k, _):
                pltpu.make_async_copy(
                    stage_ref.at[0], out_ref.at[pl.ds(k * C, C), :], sem_z,
                ).start()
                return 0
            jax.lax.fori_loop(0, T // C, zbody, 0, unroll=False)

        @pl.when(e + 1 < E)
        def _():
            issue_gather(e + 1, jax.lax.rem(e + 1, 2))

        pltpu.make_async_copy(
            x_ref.at[pl.ds(0, C), :], xe_raw.at[b], sem_in.at[b]
        ).wait()
        xs_ref[...] = (xe_raw[b] * sgate_ref[0]).astype(jnp.bfloat16)

    h = jnp.dot(xs_ref[...], fc1_ref[0].astype(jnp.bfloat16),
                preferred_element_type=jnp.float32)
    h = jnp.maximum(h + b1_ref[0], 0.0)
    contrib = jnp.dot(h.astype(jnp.bfloat16), fc2_ref[0].astype(jnp.bfloat16),
                      preferred_element_type=jnp.float32)

    @pl.when(nh == 0)
    def _():
        acc_ref[...] = contrib + b2_ref[0]

    @pl.when(nh != 0)
    def _():
        acc_ref[...] += contrib

    @pl.when(nh == NH - 1)
    def _():
        b = jax.lax.rem(e, 2)

        def drain(n, db):
            def wbody(i, _):
                pltpu.make_async_copy(
                    x_ref.at[pl.ds(0, 1), :],
                    stage_ref.at[0, pl.ds(0, 1), :],
                    sem_out.at[db],
                ).wait()
                return 0
            jax.lax.fori_loop(0, n, wbody, 0, unroll=False)

        @pl.when(e == 0)
        def _():
            def zwait(k, _):
                pltpu.make_async_copy(
                    stage_ref.at[0], out_ref.at[pl.ds(k * C, C), :], sem_z,
                ).wait()
                return 0
            jax.lax.fori_loop(0, T // C, zwait, 0, unroll=False)

        @pl.when(e >= 2)
        def _():
            drain(cnt_ref[b], b)

        stage_ref[b] = acc_ref[...] * sgate_ref[0]

        def sbody(c, cnt):
            t1 = ssrc_ref[e, c]

            def do_start():
                pltpu.make_async_copy(
                    stage_ref.at[b, pl.ds(c, 1), :],
                    out_ref.at[pl.ds(t1 - 1, 1), :],
                    sem_out.at[b],
                ).start()

            jax.lax.cond(t1 > 0, do_start, lambda: None)
            return cnt + jnp.where(t1 > 0, 1, 0)

        cnt_ref[b] = jax.lax.fori_loop(0, C, sbody, 0, unroll=False)

        @pl.when(e == E - 1)
        def _():
            drain(cnt_ref[jnp.int32(1) - b], jnp.int32(1) - b)
            drain(cnt_ref[b], b)


@jax.jit
def _moe(x, ssrc, sgate, fc1, b1, fc2, b2):
    return pl.pallas_call(
        _moe_body,
        grid=(E, NH),
        in_specs=[
            pl.BlockSpec(memory_space=pltpu.MemorySpace.HBM),
            pl.BlockSpec(memory_space=pltpu.SMEM),
            pl.BlockSpec((1, C, 1), lambda e, nh: (e, 0, 0)),
            pl.BlockSpec((1, M, BH), lambda e, nh: (e, 0, nh)),
            pl.BlockSpec((1, 1, BH), lambda e, nh: (e, 0, nh)),
            pl.BlockSpec((1, BH, M), lambda e, nh: (e, nh, 0)),
            pl.BlockSpec((1, 1, M), lambda e, nh: (e, 0, 0)),
        ],
        out_specs=pl.BlockSpec(memory_space=pltpu.MemorySpace.HBM),
        out_shape=jax.ShapeDtypeStruct((T, M), jnp.float32),
        scratch_shapes=[
            pltpu.VMEM((2, C, M), jnp.float32),
            pltpu.VMEM((C, M), jnp.bfloat16),
            pltpu.VMEM((C, M), jnp.float32),
            pltpu.VMEM((2, C, M), jnp.float32),
            pltpu.SMEM((2,), jnp.int32),
            pltpu.SemaphoreType.DMA((2,)),
            pltpu.SemaphoreType.DMA((2,)),
            pltpu.SemaphoreType.DMA,
        ],
        compiler_params=pltpu.CompilerParams(
            dimension_semantics=("arbitrary", "arbitrary"),
        ),
    )(x, ssrc, sgate, fc1, b1, fc2, b2)


def kernel(x, wg, fc1, b1, fc2, b2):
    T_, M_ = x.shape
    E_ = wg.shape[0]

    logits = x @ wg.T
    indices1_s = jnp.argmax(logits, axis=1)
    mask1 = jax.nn.one_hot(indices1_s, E_, dtype=logits.dtype)
    gates = jax.nn.softmax(logits, axis=1)
    gates1_s = jnp.sum(gates * mask1, axis=1)
    locations = jnp.cumsum(mask1, axis=0) - mask1
    locations1_s = jnp.sum(locations * mask1, axis=1).astype(jnp.int32)
    valid = locations1_s < C
    pos = indices1_s.astype(jnp.int32) * C + locations1_s
    pos_scatter = jnp.where(valid, pos, E_ * C)
    ssrc, sgate = _slots(pos_scatter, gates1_s)

    out = _moe(x, ssrc.reshape(E_, C), sgate.reshape(E_, C, 1),
               fc1, b1.reshape(E_, 1, H), fc2, b2.reshape(E_, 1, M))
    return out


# speedup vs baseline: 1.0019x; 1.0019x over previous
import functools

import jax
import jax.numpy as jnp
from jax.experimental import pallas as pl
from jax.experimental.pallas import tpu as pltpu
from jax.experimental.pallas import tpu_sc as plsc

T = 2048
M = 2048
E = 8
H = 4096
C = 256
BH = 1024
NH = H // BH

NSLOT = 2560


def _slots_body(pos_hbm, gate_hbm, ssrc_hbm, sgate_hbm,
                pos_v, gate_v, tid_v, zbuf_i, zbuf_f,
                shared_i, shared_f):
    cid = jax.lax.axis_index("c")
    sid = jax.lax.axis_index("s")

    @pl.when(cid == 0)
    def _():
        base = sid * 128
        pltpu.sync_copy(pos_hbm.at[pl.ds(base, 128)], pos_v)
        pltpu.sync_copy(gate_hbm.at[pl.ds(base, 128)], gate_v)
        for k in range(8):
            tid_v[pl.ds(k * 16, 16)] = (
                jax.lax.iota(jnp.int32, 16) + (base + k * 16 + 1))
        for k in range(10):
            zbuf_i[pl.ds(k * 16, 16)] = jnp.zeros((16,), jnp.int32)
            zbuf_f[pl.ds(k * 16, 16)] = jnp.zeros((16,), jnp.float32)
        zb = sid * (NSLOT // 16)
        pltpu.sync_copy(zbuf_i, shared_i.at[pl.ds(zb, NSLOT // 16)])
        pltpu.sync_copy(zbuf_f, shared_f.at[pl.ds(zb, NSLOT // 16)])

    plsc.subcore_barrier()

    @pl.when(cid == 0)
    def _():
        pltpu.sync_copy(tid_v, shared_i.at[pos_v], add=True)
        pltpu.sync_copy(gate_v, shared_f.at[pos_v], add=True)

    plsc.subcore_barrier()

    @pl.when(cid == 0)
    def _():
        rb = sid * 128
        pltpu.sync_copy(shared_i.at[pl.ds(rb, 128)], ssrc_hbm.at[pl.ds(rb, 128)])
        pltpu.sync_copy(shared_f.at[pl.ds(rb, 128)], sgate_hbm.at[pl.ds(rb, 128)])


@jax.jit
def _slots(pos, gate):
    f = functools.partial(
        pl.kernel,
        out_type=(jax.ShapeDtypeStruct((E * C,), jnp.int32),
                  jax.ShapeDtypeStruct((E * C,), jnp.float32)),
        mesh=plsc.VectorSubcoreMesh(core_axis_name="c", subcore_axis_name="s"),
        scratch_types=[
            pltpu.VMEM((128,), jnp.int32),
            pltpu.VMEM((128,), jnp.float32),
            pltpu.VMEM((128,), jnp.int32),
            pltpu.VMEM((NSLOT // 16,), jnp.int32),
            pltpu.VMEM((NSLOT // 16,), jnp.float32),
            pltpu.VMEM_SHARED((NSLOT,), jnp.int32),
            pltpu.VMEM_SHARED((NSLOT,), jnp.float32),
        ],
    )(_slots_body)
    return f(pos, gate)


def _moe_body(x_ref, ssrc_ref, sgate_ref, fc1_ref, b1_ref, fc2_ref, b2_ref,
              out_ref, xe_raw, xs_ref, acc_ref, stage_ref, cnt_ref,
              sem_in, sem_out, sem_z):
    e = pl.program_id(0)
    nh = pl.program_id(1)

    def issue_gather(e1, b):
        def body(c, _):
            t1 = ssrc_ref[e1, c]
            t = jnp.maximum(t1 - 1, 0)
            pltpu.make_async_copy(
                x_ref.at[pl.ds(t, 1), :],
                xe_raw.at[b, pl.ds(c, 1), :],
                sem_in.at[b],
            ).start()
            return 0
        jax.lax.fori_loop(0, C, body, 0, unroll=False)

    @pl.when(nh == 0)
    def _():
        b = jax.lax.rem(e, 2)

        @pl.when(e == 0)
        def _():
            issue_gather(0, 0)
            stage_ref[0] = jnp.zeros_like(stage_ref[0])

            def zbody(k, _):
                pltpu.make_async_copy(
                    stage_ref.at[0], out_ref.at[pl.ds(k * C, C), :], sem_z,
                ).start()
                return 0
            jax.lax.fori_loop(0, T // C, zbody, 0, unroll=False)

        @pl.when(e + 1 < E)
        def _():
            issue_gather(e + 1, jax.lax.rem(e + 1, 2))

        pltpu.make_async_copy(
            x_ref.at[pl.ds(0, C), :], xe_raw.at[b], sem_in.at[b]
        ).wait()
        xs_ref[...] = (xe_raw[b] * sgate_ref[0]).astype(jnp.bfloat16)

    h = jnp.dot(xs_ref[...], fc1_ref[0].astype(jnp.bfloat16),
                preferred_element_type=jnp.float32)
    h = jnp.maximum(h + b1_ref[0], 0.0)
    contrib = jnp.dot(h.astype(jnp.bfloat16), fc2_ref[0].astype(jnp.bfloat16),
                      preferred_element_type=jnp.float32)

    @pl.when(nh == 0)
    def _():
        acc_ref[...] = contrib + b2_ref[0]

    @pl.when(nh != 0)
    def _():
        acc_ref[...] += contrib

    @pl.when(nh == NH - 1)
    def _():
        b = jax.lax.rem(e, 2)

        def drain(n, db):
            def wbody(i, _):
                pltpu.make_async_copy(
                    x_ref.at[pl.ds(0, 1), :],
                    stage_ref.at[0, pl.ds(0, 1), :],
                    sem_out.at[db],
                ).wait()
                return 0
            jax.lax.fori_loop(0, n, wbody, 0, unroll=False)

        @pl.when(e == 0)
        def _():
            def zwait(k, _):
                pltpu.make_async_copy(
                    stage_ref.at[0], out_ref.at[pl.ds(k * C, C), :], sem_z,
                ).wait()
                return 0
            jax.lax.fori_loop(0, T // C, zwait, 0, unroll=False)

        @pl.when(e >= 2)
        def _():
            drain(cnt_ref[b], b)

        stage_ref[b] = acc_ref[...] * sgate_ref[0]

        def sbody(c, cnt):
            t1 = ssrc_ref[e, c]

            def do_start():
                pltpu.make_async_copy(
                    stage_ref.at[b, pl.ds(c, 1), :],
                    out_ref.at[pl.ds(t1 - 1, 1), :],
                    sem_out.at[b],
                ).start()

            jax.lax.cond(t1 > 0, do_start, lambda: None)
            return cnt + jnp.where(t1 > 0, 1, 0)

        cnt_ref[b] = jax.lax.fori_loop(0, C, sbody, 0, unroll=False)

        @pl.when(e == E - 1)
        def _():
            drain(cnt_ref[jnp.int32(1) - b], jnp.int32(1) - b)
            drain(cnt_ref[b], b)


@jax.jit
def _moe(x, ssrc, sgate, fc1, b1, fc2, b2):
    return pl.pallas_call(
        _moe_body,
        grid=(E, NH),
        in_specs=[
            pl.BlockSpec(memory_space=pltpu.MemorySpace.HBM),
            pl.BlockSpec(memory_space=pltpu.SMEM),
            pl.BlockSpec((1, C, 1), lambda e, nh: (e, 0, 0)),
            pl.BlockSpec((1, M, BH), lambda e, nh: (e, 0, nh)),
            pl.BlockSpec((1, 1, BH), lambda e, nh: (e, 0, nh)),
            pl.BlockSpec((1, BH, M), lambda e, nh: (e, nh, 0)),
            pl.BlockSpec((1, 1, M), lambda e, nh: (e, 0, 0)),
        ],
        out_specs=pl.BlockSpec(memory_space=pltpu.MemorySpace.HBM),
        out_shape=jax.ShapeDtypeStruct((T, M), jnp.float32),
        scratch_shapes=[
            pltpu.VMEM((2, C, M), jnp.float32),
            pltpu.VMEM((C, M), jnp.bfloat16),
            pltpu.VMEM((C, M), jnp.float32),
            pltpu.VMEM((2, C, M), jnp.float32),
            pltpu.SMEM((2,), jnp.int32),
            pltpu.SemaphoreType.DMA((2,)),
            pltpu.SemaphoreType.DMA((2,)),
            pltpu.SemaphoreType.DMA,
        ],
        compiler_params=pltpu.CompilerParams(
            dimension_semantics=("arbitrary", "arbitrary"),
        ),
    )(x, ssrc, sgate, fc1, b1, fc2, b2)


def kernel(x, wg, fc1, b1, fc2, b2):
    T_, M_ = x.shape
    E_ = wg.shape[0]

    logits = x @ wg.T
    indices1_s = jnp.argmax(logits, axis=1)
    mask1 = jax.nn.one_hot(indices1_s, E_, dtype=logits.dtype)
    gates = jax.nn.softmax(logits, axis=1)
    gates1_s = jnp.sum(gates * mask1, axis=1)
    locations = jnp.cumsum(mask1, axis=0) - mask1
    locations1_s = jnp.sum(locations * mask1, axis=1).astype(jnp.int32)
    valid = locations1_s < C
    pos = indices1_s.astype(jnp.int32) * C + locations1_s
    pos_scatter = jnp.where(valid, pos, E_ * C)
    ssrc, sgate = _slots(pos_scatter, gates1_s)

    out = _moe(x, ssrc.reshape(E_, C), sgate.reshape(E_, C, 1),
               fc1, b1.reshape(E_, 1, H), fc2, b2.reshape(E_, 1, M))
    return out
